# Initial kernel scaffold; baseline (speedup 1.0000x reference)
import jax
import jax.numpy as jnp
from jax.experimental import pallas as pl


def kernel(x):
    raise NotImplementedError("write your pallas kernel here")


# speedup vs baseline: 3.7699x; 3.7699x over previous
import jax
import jax.numpy as jnp
from jax.experimental import pallas as pl
from jax.experimental.pallas import tpu as pltpu

_BLK = 32
_R = 512


def _body(x_ref, o_ref):
    x = x_ref[0]
    c = 0.5 * (jnp.max(x, axis=1, keepdims=True)
               + jnp.min(x, axis=1, keepdims=True))
    e = jnp.exp(x - c)
    g = 1.0 / e
    for t in range(4):
        sl = slice(_BLK * t, _BLK * (t + 1))
        eb = e[:, sl]
        gb = g[:, sl]
        se = jnp.sum(eb, axis=1, keepdims=True)
        sg = jnp.sum(gb, axis=1, keepdims=True)
        o_ref[0, 0, :, sl] = eb / se
        o_ref[0, 1, :, sl] = gb / sg


@jax.jit
def kernel(x):
    B, H, S, D = x.shape
    BH = B * H
    x2 = x.reshape(BH, S, D)
    out = pl.pallas_call(
        _body,
        grid=(BH, S // _R),
        in_specs=[pl.BlockSpec((1, _R, D), lambda i, j: (i, j, 0))],
        out_specs=pl.BlockSpec((1, 2, _R, D), lambda i, j: (i, 0, j, 0)),
        out_shape=jax.ShapeDtypeStruct((BH, 2, S, D), x.dtype),
    )(x2)
    return out.reshape(B, H, 2 * S * D)


# speedup vs baseline: 5.4596x; 1.4482x over previous
import jax
import jax.numpy as jnp
from jax.experimental import pallas as pl
from jax.experimental.pallas import tpu as pltpu

_BLK = 32
_R = 512


def _body(x_ref, a_ref, o_ref):
    x = x_ref[0]
    a = a_ref[...]
    c = 0.5 * (jnp.max(x, axis=1, keepdims=True)
               + jnp.min(x, axis=1, keepdims=True))
    e = jnp.exp(x - c)
    g = 1.0 / e
    de = jax.lax.dot_general(e.astype(jnp.bfloat16), a,
                             (((1,), (0,)), ((), ())),
                             preferred_element_type=jnp.float32)
    dg = jax.lax.dot_general(g.astype(jnp.bfloat16), a,
                             (((1,), (0,)), ((), ())),
                             preferred_element_type=jnp.float32)
    o_ref[0, 0] = e / de
    o_ref[0, 1] = g / dg


@jax.jit
def kernel(x):
    B, H, S, D = x.shape
    BH = B * H
    x2 = x.reshape(BH, S, D)
    blk_id = jnp.arange(D, dtype=jnp.int32) // _BLK
    ones_blockdiag = (blk_id[:, None] == blk_id[None, :]).astype(jnp.bfloat16)
    out = pl.pallas_call(
        _body,
        grid=(BH, S // _R),
        in_specs=[
            pl.BlockSpec((1, _R, D), lambda i, j: (i, j, 0)),
            pl.BlockSpec((D, D), lambda i, j: (0, 0)),
        ],
        out_specs=pl.BlockSpec((1, 2, _R, D), lambda i, j: (i, 0, j, 0)),
        out_shape=jax.ShapeDtypeStruct((BH, 2, S, D), x.dtype),
    )(x2, ones_blockdiag)
    return out.reshape(B, H, 2 * S * D)


# speedup vs baseline: 6.9098x; 1.2656x over previous
import jax
import jax.numpy as jnp
from jax.experimental import pallas as pl
from jax.experimental.pallas import tpu as pltpu

_BLK = 32
_R = 1024


def _body(x_ref, a_ref, o_ref):
    x = x_ref[0]
    a = a_ref[...]
    c = 0.5 * (jnp.max(x, axis=1, keepdims=True)
               + jnp.min(x, axis=1, keepdims=True))
    e = jnp.exp(x - c)
    g = 1.0 / e
    de = jax.lax.dot_general(e.astype(jnp.bfloat16), a,
                             (((1,), (0,)), ((), ())),
                             preferred_element_type=jnp.float32)
    dg = jax.lax.dot_general(g.astype(jnp.bfloat16), a,
                             (((1,), (0,)), ((), ())),
                             preferred_element_type=jnp.float32)
    o_ref[0, 0] = e / de
    o_ref[0, 1] = g / dg


@jax.jit
def kernel(x):
    B, H, S, D = x.shape
    BH = B * H
    x2 = x.reshape(BH, S, D)
    blk_id = jnp.arange(D, dtype=jnp.int32) // _BLK
    ones_blockdiag = (blk_id[:, None] == blk_id[None, :]).astype(jnp.bfloat16)
    out = pl.pallas_call(
        _body,
        grid=(BH, S // _R),
        in_specs=[
            pl.BlockSpec((1, _R, D), lambda i, j: (i, j, 0)),
            pl.BlockSpec((D, D), lambda i, j: (0, 0)),
        ],
        out_specs=pl.BlockSpec((1, 2, _R, D), lambda i, j: (i, 0, j, 0)),
        out_shape=jax.ShapeDtypeStruct((BH, 2, S, D), x.dtype),
    )(x2, ones_blockdiag)
    return out.reshape(B, H, 2 * S * D)


# speedup vs baseline: 8.0340x; 1.1627x over previous
import jax
import jax.numpy as jnp
from jax.experimental import pallas as pl
from jax.experimental.pallas import tpu as pltpu

_BLK = 32
_R = 2048


def _body(x_ref, a_ref, o_ref):
    x = x_ref[0]
    a = a_ref[...]
    c = 0.5 * (jnp.max(x, axis=1, keepdims=True)
               + jnp.min(x, axis=1, keepdims=True))
    e = jnp.exp(x - c)
    g = 1.0 / e
    de = jax.lax.dot_general(e.astype(jnp.bfloat16), a,
                             (((1,), (0,)), ((), ())),
                             preferred_element_type=jnp.float32)
    dg = jax.lax.dot_general(g.astype(jnp.bfloat16), a,
                             (((1,), (0,)), ((), ())),
                             preferred_element_type=jnp.float32)
    o_ref[0, 0] = e / de
    o_ref[0, 1] = g / dg


@jax.jit
def kernel(x):
    B, H, S, D = x.shape
    BH = B * H
    x2 = x.reshape(BH, S, D)
    blk_id = jnp.arange(D, dtype=jnp.int32) // _BLK
    ones_blockdiag = (blk_id[:, None] == blk_id[None, :]).astype(jnp.bfloat16)
    out = pl.pallas_call(
        _body,
        grid=(BH, S // _R),
        in_specs=[
            pl.BlockSpec((1, _R, D), lambda i, j: (i, j, 0)),
            pl.BlockSpec((D, D), lambda i, j: (0, 0)),
        ],
        out_specs=pl.BlockSpec((1, 2, _R, D), lambda i, j: (i, 0, j, 0)),
        out_shape=jax.ShapeDtypeStruct((BH, 2, S, D), x.dtype),
    )(x2, ones_blockdiag)
    return out.reshape(B, H, 2 * S * D)


# speedup vs baseline: 8.7412x; 1.0880x over previous
import jax
import jax.numpy as jnp
from jax.experimental import pallas as pl
from jax.experimental.pallas import tpu as pltpu

_BLK = 32
_R = 4096


def _body(x_ref, a_ref, o_ref):
    x = x_ref[0]
    a = a_ref[...]
    c = 0.5 * (jnp.max(x, axis=1, keepdims=True)
               + jnp.min(x, axis=1, keepdims=True))
    e = jnp.exp(x - c)
    g = 1.0 / e
    de = jax.lax.dot_general(e.astype(jnp.bfloat16), a,
                             (((1,), (0,)), ((), ())),
                             preferred_element_type=jnp.float32)
    dg = jax.lax.dot_general(g.astype(jnp.bfloat16), a,
                             (((1,), (0,)), ((), ())),
                             preferred_element_type=jnp.float32)
    o_ref[0, 0] = e / de
    o_ref[0, 1] = g / dg


@jax.jit
def kernel(x):
    B, H, S, D = x.shape
    BH = B * H
    x2 = x.reshape(BH, S, D)
    blk_id = jnp.arange(D, dtype=jnp.int32) // _BLK
    ones_blockdiag = (blk_id[:, None] == blk_id[None, :]).astype(jnp.bfloat16)
    out = pl.pallas_call(
        _body,
        grid=(BH, S // _R),
        in_specs=[
            pl.BlockSpec((1, _R, D), lambda i, j: (i, j, 0)),
            pl.BlockSpec((D, D), lambda i, j: (0, 0)),
        ],
        out_specs=pl.BlockSpec((1, 2, _R, D), lambda i, j: (i, 0, j, 0)),
        out_shape=jax.ShapeDtypeStruct((BH, 2, S, D), x.dtype),
    )(x2, ones_blockdiag)
    return out.reshape(B, H, 2 * S * D)


# speedup vs baseline: 9.1312x; 1.0446x over previous
import jax
import jax.numpy as jnp
from jax.experimental import pallas as pl
from jax.experimental.pallas import tpu as pltpu

_BLK = 32
_G = 2


def _body(x_ref, a_ref, o_ref):
    G, S, D = x_ref.shape
    x = x_ref[...].reshape(G * S, D)
    a = a_ref[...]
    c = 0.5 * (jnp.max(x, axis=1, keepdims=True)
               + jnp.min(x, axis=1, keepdims=True))
    e = jnp.exp(x - c)
    g = 1.0 / e
    de = jax.lax.dot_general(e.astype(jnp.bfloat16), a,
                             (((1,), (0,)), ((), ())),
                             preferred_element_type=jnp.float32)
    dg = jax.lax.dot_general(g.astype(jnp.bfloat16), a,
                             (((1,), (0,)), ((), ())),
                             preferred_element_type=jnp.float32)
    o_ref[:, 0] = (e / de).reshape(G, S, D)
    o_ref[:, 1] = (g / dg).reshape(G, S, D)


@jax.jit
def kernel(x):
    B, H, S, D = x.shape
    BH = B * H
    x2 = x.reshape(BH, S, D)
    blk_id = jnp.arange(D, dtype=jnp.int32) // _BLK
    ones_blockdiag = (blk_id[:, None] == blk_id[None, :]).astype(jnp.bfloat16)
    out = pl.pallas_call(
        _body,
        grid=(BH // _G,),
        in_specs=[
            pl.BlockSpec((_G, S, D), lambda i: (i, 0, 0)),
            pl.BlockSpec((D, D), lambda i: (0, 0)),
        ],
        out_specs=pl.BlockSpec((_G, 2, S, D), lambda i: (i, 0, 0, 0)),
        out_shape=jax.ShapeDtypeStruct((BH, 2, S, D), x.dtype),
    )(x2, ones_blockdiag)
    return out.reshape(B, H, 2 * S * D)


# speedup vs baseline: 9.1622x; 1.0034x over previous
import jax
import jax.numpy as jnp
from jax.experimental import pallas as pl
from jax.experimental.pallas import tpu as pltpu

_BLK = 32
_G = 4


def _body(x_ref, a_ref, o_ref):
    G, S, D = x_ref.shape
    x = x_ref[...].reshape(G * S, D)
    a = a_ref[...]
    c = 0.5 * (jnp.max(x, axis=1, keepdims=True)
               + jnp.min(x, axis=1, keepdims=True))
    e = jnp.exp(x - c)
    g = 1.0 / e
    de = jax.lax.dot_general(e.astype(jnp.bfloat16), a,
                             (((1,), (0,)), ((), ())),
                             preferred_element_type=jnp.float32)
    dg = jax.lax.dot_general(g.astype(jnp.bfloat16), a,
                             (((1,), (0,)), ((), ())),
                             preferred_element_type=jnp.float32)
    o_ref[:, 0] = (e / de).reshape(G, S, D)
    o_ref[:, 1] = (g / dg).reshape(G, S, D)


@jax.jit
def kernel(x):
    B, H, S, D = x.shape
    BH = B * H
    x2 = x.reshape(BH, S, D)
    blk_id = jnp.arange(D, dtype=jnp.int32) // _BLK
    ones_blockdiag = (blk_id[:, None] == blk_id[None, :]).astype(jnp.bfloat16)
    out = pl.pallas_call(
        _body,
        grid=(BH // _G,),
        in_specs=[
            pl.BlockSpec((_G, S, D), lambda i: (i, 0, 0)),
            pl.BlockSpec((D, D), lambda i: (0, 0)),
        ],
        out_specs=pl.BlockSpec((_G, 2, S, D), lambda i: (i, 0, 0, 0)),
        out_shape=jax.ShapeDtypeStruct((BH, 2, S, D), x.dtype),
    )(x2, ones_blockdiag)
    return out.reshape(B, H, 2 * S * D)
